# baseline (speedup 1.0000x reference)
import jax
import jax.numpy as jnp
from jax.experimental import pallas as pl
from jax.experimental.pallas import tpu as pltpu

D_MODEL = 768
NUM_EXPERTS = 64
N_TOKENS = 32768
BT = 4096
NSTEPS = N_TOKENS // BT


def _router_body(x_ref, w_ref, b_ref, logits_ref, probs_ref,
                 lbuf, pbuf, lsem, psem):
    i = pl.program_id(0)
    slot = jax.lax.rem(i, 2)

    def _lcopy(step, s):
        return pltpu.make_async_copy(
            lbuf.at[s], logits_ref.at[pl.ds(step * BT, BT), :], lsem.at[s]
        )

    def _pcopy(step, s):
        return pltpu.make_async_copy(
            pbuf.at[s], probs_ref.at[pl.ds(step * BT, BT), :], psem.at[s]
        )

    @pl.when(i >= 2)
    def _():
        _lcopy(i - 2, slot).wait()
        _pcopy(i - 2, slot).wait()

    logits = jax.lax.dot_general(
        x_ref[...], w_ref[...], (((1,), (1,)), ((), ())),
        preferred_element_type=jnp.float32,
    )
    logits = logits + b_ref[...]
    lbuf[slot] = logits
    m = jnp.max(logits, axis=-1, keepdims=True)
    e = jnp.exp(logits - m)
    pbuf[slot] = e / jnp.sum(e, axis=-1, keepdims=True)

    _lcopy(i, slot).start()
    _pcopy(i, slot).start()

    @pl.when(i == NSTEPS - 1)
    def _():
        _lcopy(i - 1, 1 - slot).wait()
        _pcopy(i - 1, 1 - slot).wait()
        _lcopy(i, slot).wait()
        _pcopy(i, slot).wait()


def kernel(x, W, b):
    b2 = b.reshape(1, NUM_EXPERTS)
    out_shape = (
        jax.ShapeDtypeStruct((N_TOKENS, NUM_EXPERTS), jnp.float32),
        jax.ShapeDtypeStruct((N_TOKENS, NUM_EXPERTS), jnp.float32),
    )
    logits, probs = pl.pallas_call(
        _router_body,
        grid=(NSTEPS,),
        in_specs=[
            pl.BlockSpec((BT, D_MODEL), lambda i: (i, 0)),
            pl.BlockSpec((NUM_EXPERTS, D_MODEL), lambda i: (0, 0)),
            pl.BlockSpec((1, NUM_EXPERTS), lambda i: (0, 0)),
        ],
        out_specs=(
            pl.BlockSpec(memory_space=pltpu.MemorySpace.HBM),
            pl.BlockSpec(memory_space=pltpu.MemorySpace.HBM),
        ),
        out_shape=out_shape,
        scratch_shapes=[
            pltpu.VMEM((2, BT, NUM_EXPERTS), jnp.float32),
            pltpu.VMEM((2, BT, NUM_EXPERTS), jnp.float32),
            pltpu.SemaphoreType.DMA((2,)),
            pltpu.SemaphoreType.DMA((2,)),
        ],
        compiler_params=pltpu.CompilerParams(
            dimension_semantics=("arbitrary",),
        ),
    )(x, W, b2)
    return (logits, probs)
